# baseline (speedup 1.0000x reference)
import functools

import jax
import jax.numpy as jnp
from jax import lax
from jax.experimental import pallas as pl
from jax.experimental.pallas import tpu as pltpu
from jax.experimental.pallas import tpu_sc as plsc

K = 32
_BISECT_A = 9
_BISECT_B = 12


def _encode_select_body(x_ref, b_dec_ref, w_ref, b_enc_ref, f_ref):
    a = (x_ref[...] - b_dec_ref[...]).astype(jnp.bfloat16)
    pre = lax.dot_general(
        a, w_ref[...], (((1,), (0,)), ((), ())),
        preferred_element_type=jnp.float32,
    )
    pre = pre + b_enc_ref[...]

    hi = jnp.max(pre, axis=1, keepdims=True)
    lo = jnp.zeros_like(hi)
    kf = jnp.float32(K)
    bt, nf = pre.shape

    def count_ge(data, mid, one):
        acc = jnp.where(data >= mid, one, one * 0)
        while acc.shape[1] > 128:
            h = acc.shape[1] // 2
            acc = acc[:, :h] + acc[:, h:]
        return jnp.sum(acc.astype(jnp.float32), axis=1, keepdims=True)

    q16 = lax.bitcast_convert_type(
        jnp.bitwise_and(lax.bitcast_convert_type(pre, jnp.int32),
                        jnp.int32(-65536)),
        jnp.float32).astype(jnp.bfloat16)
    one16 = jnp.ones((), jnp.bfloat16)
    for _ in range(_BISECT_A):
        mid16 = ((lo + hi) * 0.5).astype(jnp.bfloat16)
        c = count_ge(q16, mid16, one16)
        big = c >= kf
        midf = mid16.astype(jnp.float32)
        lo = jnp.where(big, midf, lo)
        hi = jnp.where(big, hi, midf)

    one32 = jnp.ones((), jnp.float32)
    for _ in range(_BISECT_B):
        mid = (lo + hi) * 0.5
        c = count_ge(pre, mid, one32)
        big = c >= kf
        lo = jnp.where(big, mid, lo)
        hi = jnp.where(big, hi, mid)
    f_ref[...] = jnp.where(pre >= lo, jnp.maximum(pre, 0.0), 0.0)


def _encode_select(x, b_dec_row, w_enc16, b_enc_row, block_t):
    n, d = x.shape
    nf = w_enc16.shape[1]
    return pl.pallas_call(
        _encode_select_body,
        grid=(n // block_t,),
        in_specs=[
            pl.BlockSpec((block_t, d), lambda i: (i, 0)),
            pl.BlockSpec((1, d), lambda i: (0, 0)),
            pl.BlockSpec((d, nf), lambda i: (0, 0)),
            pl.BlockSpec((1, nf), lambda i: (0, 0)),
        ],
        out_specs=pl.BlockSpec((block_t, nf), lambda i: (i, 0)),
        out_shape=jax.ShapeDtypeStruct((n, nf), jnp.float32),
    )(x, b_dec_row, w_enc16, b_enc_row)


def _encode_thresh_body(x_ref, b_dec_ref, w_ref, b_enc_ref, pre_ref, t_ref):
    a = (x_ref[...] - b_dec_ref[...]).astype(jnp.bfloat16)
    pre = lax.dot_general(
        a, w_ref[...], (((1,), (0,)), ((), ())),
        preferred_element_type=jnp.float32,
    )
    pre = pre + b_enc_ref[...]

    hi = jnp.max(pre, axis=1, keepdims=True)
    lo = jnp.zeros_like(hi)
    kf = jnp.float32(K)

    def count_ge(data, mid, one):
        acc = jnp.where(data >= mid, one, one * 0)
        while acc.shape[1] > 128:
            h = acc.shape[1] // 2
            acc = acc[:, :h] + acc[:, h:]
        return jnp.sum(acc.astype(jnp.float32), axis=1, keepdims=True)

    q16 = lax.bitcast_convert_type(
        jnp.bitwise_and(lax.bitcast_convert_type(pre, jnp.int32),
                        jnp.int32(-65536)),
        jnp.float32).astype(jnp.bfloat16)
    one16 = jnp.ones((), jnp.bfloat16)
    for _ in range(_BISECT_A):
        mid16 = ((lo + hi) * 0.5).astype(jnp.bfloat16)
        big = count_ge(q16, mid16, one16) >= kf
        midf = mid16.astype(jnp.float32)
        lo = jnp.where(big, midf, lo)
        hi = jnp.where(big, hi, midf)
    one32 = jnp.ones((), jnp.float32)
    for _ in range(_BISECT_B):
        mid = (lo + hi) * 0.5
        big = count_ge(pre, mid, one32) >= kf
        lo = jnp.where(big, mid, lo)
        hi = jnp.where(big, hi, mid)
    pre_ref[...] = pre
    t_ref[...] = jnp.broadcast_to(lo, (pre.shape[0], 16))


def _encode_thresh(x, b_dec_row, w_enc16, b_enc_row, block_t):
    n, d = x.shape
    nf = w_enc16.shape[1]
    return pl.pallas_call(
        _encode_thresh_body,
        grid=(n // block_t,),
        in_specs=[
            pl.BlockSpec((block_t, d), lambda i: (i, 0)),
            pl.BlockSpec((1, d), lambda i: (0, 0)),
            pl.BlockSpec((d, nf), lambda i: (0, 0)),
            pl.BlockSpec((1, nf), lambda i: (0, 0)),
        ],
        out_specs=[
            pl.BlockSpec((block_t, nf), lambda i: (i, 0)),
            pl.BlockSpec((block_t, 16), lambda i: (i, 0)),
        ],
        out_shape=[
            jax.ShapeDtypeStruct((n, nf), jnp.float32),
            jax.ShapeDtypeStruct((n, 16), jnp.float32),
        ],
    )(x, b_dec_row, w_enc16, b_enc_row)


_SC_CHUNK = 4


def _sc_mask_write(pre, t_rep):
    n, nf = pre.shape
    info = plsc.get_sparse_core_info()
    nc, ns = info.num_cores, info.num_subcores
    nw = nc * ns
    rows_w = n // nw
    ch = _SC_CHUNK
    mesh = plsc.VectorSubcoreMesh(core_axis_name="c", subcore_axis_name="s")

    @functools.partial(
        pl.kernel, mesh=mesh,
        out_type=jax.ShapeDtypeStruct((n, nf), jnp.float32),
        scratch_types=[
            pltpu.VMEM((ch, nf), jnp.float32),
            pltpu.VMEM((ch, nf), jnp.float32),
            pltpu.VMEM((ch, 16), jnp.float32),
        ],
    )
    def k(pre_hbm, t_hbm, f_hbm, inb, outb, tb):
        wid = lax.axis_index("s") * nc + lax.axis_index("c")
        base = wid * rows_w

        def chunk(ci, carry):
            r0 = base + ci * ch
            pltpu.sync_copy(pre_hbm.at[pl.ds(r0, ch)], inb)
            pltpu.sync_copy(t_hbm.at[pl.ds(r0, ch)], tb)
            for ri in range(ch):
                tv = tb[ri, pl.ds(0, 16)]

                def vloop(j, c2):
                    v = inb[ri, pl.ds(j * 16, 16)]
                    outb[ri, pl.ds(j * 16, 16)] = jnp.where(
                        v >= tv, jnp.maximum(v, 0.0), 0.0)
                    return c2

                lax.fori_loop(0, nf // 16, vloop, 0, unroll=8)
            pltpu.sync_copy(outb, f_hbm.at[pl.ds(r0, ch)])
            return carry

        lax.fori_loop(0, rows_w // ch, chunk, 0)

    return k(pre, t_rep)


def _decode_body(f_ref, w_ref, b_dec_ref, out_ref):
    out_ref[...] = lax.dot_general(
        f_ref[...].astype(jnp.bfloat16), w_ref[...],
        (((1,), (0,)), ((), ())),
        preferred_element_type=jnp.float32,
    ) + b_dec_ref[...]


def _decode(f, w_dec16, b_dec_row, block_t):
    n, nf = f.shape
    d = w_dec16.shape[1]
    return pl.pallas_call(
        _decode_body,
        grid=(n // block_t,),
        in_specs=[
            pl.BlockSpec((block_t, nf), lambda i: (i, 0)),
            pl.BlockSpec((nf, d), lambda i: (0, 0)),
            pl.BlockSpec((1, d), lambda i: (0, 0)),
        ],
        out_specs=pl.BlockSpec((block_t, d), lambda i: (i, 0)),
        out_shape=jax.ShapeDtypeStruct((n, d), jnp.float32),
        compiler_params=pltpu.CompilerParams(
            dimension_semantics=("arbitrary",),
        ),
    )(f, w_dec16, b_dec_row)


def kernel(x, b_dec, W_enc, b_enc, W_dec):
    n, d = x.shape
    nf = W_enc.shape[1]
    w_enc16 = W_enc.astype(jnp.bfloat16)
    w_dec16 = W_dec.astype(jnp.bfloat16)
    b_dec_row = b_dec.reshape(1, d)
    b_enc_row = b_enc.reshape(1, nf)
    block_t = min(256, n)
    pre, t_rep = _encode_thresh(x, b_dec_row, w_enc16, b_enc_row, block_t)
    f = _sc_mask_write(pre, t_rep)
    recon = _decode(f, w_dec16, b_dec_row, min(256, n))
    return recon, f
